# baseline (speedup 1.0000x reference)
import functools

import jax
import jax.numpy as jnp
from jax import lax
from jax.experimental import pallas as pl
from jax.experimental.pallas import tpu as pltpu
from jax.experimental.pallas import tpu_sc as plsc

B = 128
N = 32768
K = 8
LANES = 16
NUM_CORES = 2
NUM_SUBCORES = 16
NUM_TILES = NUM_CORES * NUM_SUBCORES
COLS_PER_TILE = N // NUM_TILES
W = 256
CHUNKS = COLS_PER_TILE // W
NBUF = 3
GROUPS = W // LANES
ROW_BLOCKS = B // K

_SORT8 = ((0, 1), (2, 3), (4, 5), (6, 7),
          (0, 2), (1, 3), (4, 6), (5, 7),
          (1, 2), (5, 6),
          (0, 4), (1, 5), (2, 6), (3, 7),
          (2, 4), (3, 5),
          (1, 2), (3, 4), (5, 6))
_BITONIC8 = ((0, 4), (1, 5), (2, 6), (3, 7),
             (0, 2), (1, 3), (4, 6), (5, 7),
             (0, 1), (2, 3), (4, 5), (6, 7))


def _net_desc(vals, net):
    vals = list(vals)
    for i, j in net:
        hi = jnp.maximum(vals[i], vals[j])
        lo = jnp.minimum(vals[i], vals[j])
        vals[i], vals[j] = hi, lo
    return vals


def _topk_mask_body(a_hbm, out_hbm, buf_0, buf_1, buf_2,
                    sem_i0, sem_i1, sem_i2, sem_o0, sem_o1, sem_o2):
    cc = lax.axis_index("c")
    sid = lax.axis_index("s")
    cbase = (cc * NUM_SUBCORES + sid) * COLS_PER_TILE
    zero = jnp.zeros((LANES,), jnp.float32)

    bufs = (buf_0, buf_1, buf_2)
    in_sems = (sem_i0, sem_i1, sem_i2)
    out_sems = (sem_o0, sem_o1, sem_o2)

    def compute_chunk(ibuf):
        def group_body(g, _):
            col = g * LANES

            run = _net_desc(
                [ibuf[j, pl.ds(col, LANES)] for j in range(K)], _SORT8)

            def blk_body(rb, run):
                s = _net_desc(
                    [ibuf[rb * K + j, pl.ds(col, LANES)] for j in range(K)],
                    _SORT8)
                merged = [jnp.maximum(run[i], s[K - 1 - i]) for i in range(K)]
                return tuple(_net_desc(merged, _BITONIC8))

            run = lax.fori_loop(1, ROW_BLOCKS, blk_body, tuple(run))
            thr = run[K - 1]

            def mask_body(rb, _):
                for j in range(K):
                    r = rb * K + j
                    v = ibuf[r, pl.ds(col, LANES)]
                    ibuf[r, pl.ds(col, LANES)] = jnp.where(v >= thr, v, zero)
                return 0

            lax.fori_loop(0, ROW_BLOCKS, mask_body, 0)
            return 0

        lax.fori_loop(0, GROUPS, group_body, 0)

    in_h = [None] * CHUNKS
    out_h = [None] * CHUNKS
    for c in range(min(NBUF, CHUNKS)):
        in_h[c] = pltpu.async_copy(
            a_hbm.at[:, pl.ds(cbase + c * W, W)], bufs[c % NBUF],
            in_sems[c % NBUF])
    waited = set()
    for c in range(CHUNKS):
        in_h[c].wait()
        compute_chunk(bufs[c % NBUF])
        out_h[c] = pltpu.async_copy(
            bufs[c % NBUF], out_hbm.at[:, pl.ds(cbase + c * W, W)],
            out_sems[c % NBUF])
        nxt = c + NBUF - 1
        if NBUF <= nxt < CHUNKS:
            out_h[nxt - NBUF].wait()
            waited.add(nxt - NBUF)
            in_h[nxt] = pltpu.async_copy(
                a_hbm.at[:, pl.ds(cbase + nxt * W, W)], bufs[nxt % NBUF],
                in_sems[nxt % NBUF])
    for c in range(CHUNKS):
        if c not in waited:
            out_h[c].wait()


@jax.jit
def _topk_mask(activations):
    mesh = plsc.VectorSubcoreMesh(core_axis_name="c", subcore_axis_name="s")
    f = functools.partial(
        pl.kernel,
        out_type=jax.ShapeDtypeStruct((B, N), jnp.float32),
        mesh=mesh,
        scratch_types=[
            pltpu.VMEM((B, W), jnp.float32),
            pltpu.VMEM((B, W), jnp.float32),
            pltpu.VMEM((B, W), jnp.float32),
            pltpu.SemaphoreType.DMA,
            pltpu.SemaphoreType.DMA,
            pltpu.SemaphoreType.DMA,
            pltpu.SemaphoreType.DMA,
            pltpu.SemaphoreType.DMA,
            pltpu.SemaphoreType.DMA,
        ],
    )(_topk_mask_body)
    return f(activations)


def _relayout_body(x_ref, o_ref):
    blk = x_ref.shape[1]
    o_ref[...] = x_ref[...].reshape(B, blk // 128, 128)


@jax.jit
def _relayout(x):
    blk = 32768
    return pl.pallas_call(
        _relayout_body,
        grid=(N // blk,),
        in_specs=[pl.BlockSpec((B, blk), lambda i: (0, i))],
        out_specs=pl.BlockSpec((B, blk // 128, 128), lambda i: (0, i, 0)),
        out_shape=jax.ShapeDtypeStruct((B, N // 128, 128), jnp.float32),
    )(x)


def kernel(activations):
    return _relayout(_topk_mask(activations)).reshape(B, N, 1, 1)


# speedup vs baseline: 1.0639x; 1.0639x over previous
import functools

import jax
import jax.numpy as jnp
from jax import lax
from jax.experimental import pallas as pl
from jax.experimental.pallas import tpu as pltpu
from jax.experimental.pallas import tpu_sc as plsc

B = 128
N = 32768
K = 8
LANES = 16
NUM_CORES = 2
NUM_SUBCORES = 16
NUM_TILES = NUM_CORES * NUM_SUBCORES
COLS_PER_TILE = N // NUM_TILES
W = 256
CHUNKS = COLS_PER_TILE // W
NBUF = 3
GROUPS = W // LANES
ROW_BLOCKS = B // K

_SORT8 = ((0, 1), (2, 3), (4, 5), (6, 7),
          (0, 2), (1, 3), (4, 6), (5, 7),
          (1, 2), (5, 6),
          (0, 4), (1, 5), (2, 6), (3, 7),
          (2, 4), (3, 5),
          (1, 2), (3, 4), (5, 6))
_BITONIC8 = ((0, 4), (1, 5), (2, 6), (3, 7),
             (0, 2), (1, 3), (4, 6), (5, 7),
             (0, 1), (2, 3), (4, 5), (6, 7))


def _net_desc(vals, net):
    vals = list(vals)
    for i, j in net:
        hi = jnp.maximum(vals[i], vals[j])
        lo = jnp.minimum(vals[i], vals[j])
        vals[i], vals[j] = hi, lo
    return vals


def _topk_mask_body(a_hbm, out_hbm, buf_0, buf_1, buf_2,
                    sem_i0, sem_i1, sem_i2, sem_o0, sem_o1, sem_o2):
    cc = lax.axis_index("c")
    sid = lax.axis_index("s")
    cbase = (cc * NUM_SUBCORES + sid) * COLS_PER_TILE
    zero = jnp.zeros((LANES,), jnp.float32)

    bufs = (buf_0, buf_1, buf_2)
    in_sems = (sem_i0, sem_i1, sem_i2)
    out_sems = (sem_o0, sem_o1, sem_o2)

    def compute_chunk(ibuf):
        def group_body(g, _):
            col = g * LANES

            run = _net_desc(
                [ibuf[j, pl.ds(col, LANES)] for j in range(K)], _SORT8)

            def blk_body(rb, run):
                s = _net_desc(
                    [ibuf[rb * K + j, pl.ds(col, LANES)] for j in range(K)],
                    _SORT8)
                merged = [jnp.maximum(run[i], s[K - 1 - i]) for i in range(K)]
                return tuple(_net_desc(merged, _BITONIC8))

            run = lax.fori_loop(1, ROW_BLOCKS, blk_body, tuple(run))
            thr = run[K - 1]

            def mask_body(rb, _):
                for j in range(K):
                    r = rb * K + j
                    v = ibuf[r, pl.ds(col, LANES)]
                    ibuf[r, pl.ds(col, LANES)] = jnp.where(v >= thr, v, zero)
                return 0

            lax.fori_loop(0, ROW_BLOCKS, mask_body, 0)
            return 0

        lax.fori_loop(0, GROUPS, group_body, 0)

    in_h = [None] * CHUNKS
    out_h = [None] * CHUNKS
    for c in range(min(NBUF, CHUNKS)):
        in_h[c] = pltpu.async_copy(
            a_hbm.at[:, pl.ds(cbase + c * W, W)], bufs[c % NBUF],
            in_sems[c % NBUF])
    waited = set()
    for c in range(CHUNKS):
        in_h[c].wait()
        compute_chunk(bufs[c % NBUF])
        out_h[c] = pltpu.async_copy(
            bufs[c % NBUF], out_hbm.at[:, pl.ds(cbase + c * W, W)],
            out_sems[c % NBUF])
        nxt = c + NBUF - 1
        if NBUF <= nxt < CHUNKS:
            out_h[nxt - NBUF].wait()
            waited.add(nxt - NBUF)
            in_h[nxt] = pltpu.async_copy(
                a_hbm.at[:, pl.ds(cbase + nxt * W, W)], bufs[nxt % NBUF],
                in_sems[nxt % NBUF])
    for c in range(CHUNKS):
        if c not in waited:
            out_h[c].wait()


@jax.jit
def _topk_mask(activations):
    mesh = plsc.VectorSubcoreMesh(core_axis_name="c", subcore_axis_name="s")
    f = functools.partial(
        pl.kernel,
        out_type=jax.ShapeDtypeStruct((B, N), jnp.float32),
        mesh=mesh,
        scratch_types=[
            pltpu.VMEM((B, W), jnp.float32),
            pltpu.VMEM((B, W), jnp.float32),
            pltpu.VMEM((B, W), jnp.float32),
            pltpu.SemaphoreType.DMA,
            pltpu.SemaphoreType.DMA,
            pltpu.SemaphoreType.DMA,
            pltpu.SemaphoreType.DMA,
            pltpu.SemaphoreType.DMA,
            pltpu.SemaphoreType.DMA,
        ],
    )(_topk_mask_body)
    return f(activations)


def _relayout_body(x_ref, o_ref):
    blk = x_ref.shape[1]
    o_ref[...] = x_ref[...].reshape(B, blk // 128, 128)


@jax.jit
def _relayout(x):
    blk = 16384
    return pl.pallas_call(
        _relayout_body,
        grid=(N // blk,),
        in_specs=[pl.BlockSpec((B, blk), lambda i: (0, i))],
        out_specs=pl.BlockSpec((B, blk // 128, 128), lambda i: (0, i, 0)),
        out_shape=jax.ShapeDtypeStruct((B, N // 128, 128), jnp.float32),
    )(x)


def kernel(activations):
    return _relayout(_topk_mask(activations)).reshape(B, N, 1, 1)
